# baseline (speedup 1.0000x reference)
import functools

import jax
import jax.numpy as jnp
from jax import lax
from jax.experimental import pallas as pl
from jax.experimental.pallas import tpu as pltpu
from jax.experimental.pallas import tpu_sc as plsc


def _sc_gather(features, idx_flat):
    n_rows = idx_flat.shape[0]
    feat_dim = features.shape[1]
    info = plsc.get_sparse_core_info()
    n_workers = info.num_cores * info.num_subcores
    assert n_rows % n_workers == 0
    per_worker = n_rows // n_workers
    chunk = 128
    while per_worker % chunk:
        chunk //= 2
    n_chunks = per_worker // chunk
    num_cores = info.num_cores

    mesh = plsc.VectorSubcoreMesh(core_axis_name="c", subcore_axis_name="s")

    @functools.partial(
        pl.kernel,
        out_type=jax.ShapeDtypeStruct((n_rows, feat_dim), features.dtype),
        mesh=mesh,
        scratch_types=[
            pltpu.VMEM((2, chunk), jnp.int32),
            pltpu.VMEM((2, chunk, feat_dim), jnp.float32),
            pltpu.SemaphoreType.DMA,
            pltpu.SemaphoreType.DMA,
        ],
    )
    def gather_kernel(table_hbm, idx_hbm, out_hbm, idx_v, rows_v, sem0, sem1):
        wid = lax.axis_index("s") * num_cores + lax.axis_index("c")
        base = wid * per_worker
        sems = [sem0, sem1]

        def start(c):
            buf = c % 2
            off = base + c * chunk
            pltpu.sync_copy(idx_hbm.at[pl.ds(off, chunk)], idx_v.at[buf])
            return pltpu.async_copy(
                table_hbm.at[idx_v.at[buf]], rows_v.at[buf], sems[buf])

        pending = start(0)
        for c in range(n_chunks):
            nxt = start(c + 1) if c + 1 < n_chunks else None
            pending.wait()
            pltpu.sync_copy(rows_v.at[c % 2],
                            out_hbm.at[pl.ds(base + c * chunk, chunk)])
            pending = nxt

    return gather_kernel(features, idx_flat)


def _attn_body(emb_ref, w11_ref, w12_ref, w21_ref, w22_ref, w31_ref,
               w32_ref, out_ref):
    x = emb_ref[...]
    k_fan, b_blk, f_dim = x.shape
    xT = jnp.transpose(x.reshape(k_fan * b_blk, f_dim))

    def matT(w_ref, rhs):
        return jax.lax.dot_general(w_ref[...], rhs, (((0,), (0,)), ((), ())))

    u2T = jnp.tanh(matT(w11_ref, xT))
    g = jax.lax.dot_general(
        w22_ref[...], w12_ref[...], (((1,), (1,)), ((), ())))
    t2T = g @ u2T
    t3T = jnp.tanh(matT(w21_ref, xT))
    t4T = matT(w32_ref, jnp.tanh(matT(w31_ref, xT)))

    t2s = [t2T[:, k * b_blk:(k + 1) * b_blk] for k in range(k_fan)]
    t3s = [t3T[:, k * b_blk:(k + 1) * b_blk] for k in range(k_fan)]
    t4s = [t4T[:, k * b_blk:(k + 1) * b_blk] for k in range(k_fan)]

    w = [jnp.zeros((b_blk,), jnp.float32) for _ in range(k_fan)]
    for k in range(k_fan):
        s = [jnp.sum(t2s[k] * t3s[j], axis=0) for j in range(k_fan)]
        m = s[0]
        for j in range(1, k_fan):
            m = jnp.maximum(m, s[j])
        e = [jnp.exp(s[j] - m) for j in range(k_fan)]
        tot = e[0]
        for j in range(1, k_fan):
            tot = tot + e[j]
        inv = 1.0 / tot
        for j in range(k_fan):
            w[j] = w[j] + e[j] * inv

    outT = w[0][None, :] * t4s[0]
    for j in range(1, k_fan):
        outT = outT + w[j][None, :] * t4s[j]
    out_ref[...] = jnp.transpose(outT)


def _attn_call(emb_t, W11, W12, W21, W22, W31, W32, b_blk=512):
    k_fan, batch, f_dim = emb_t.shape
    h_dim = W12.shape[1]
    grid = (batch // b_blk,)
    wspec = pl.BlockSpec((f_dim, h_dim), lambda i: (0, 0))
    return pl.pallas_call(
        _attn_body,
        grid=grid,
        in_specs=[
            pl.BlockSpec((k_fan, b_blk, f_dim), lambda i: (0, i, 0)),
            wspec, wspec, wspec, wspec, wspec, wspec,
        ],
        out_specs=pl.BlockSpec((b_blk, h_dim), lambda i: (i, 0)),
        out_shape=jax.ShapeDtypeStruct((batch, h_dim), jnp.float32),
    )(emb_t, W11, W12, W21, W22, W31, W32)


def kernel(nodes, to_neighs, features, W11, W12, W21, W22, W31, W32):
    del nodes
    batch, k_fan = to_neighs.shape
    f_dim = features.shape[1]
    parts = [batch // 4, 3 * batch // 8, 3 * batch // 8]
    embs, off = [], 0
    for b_part in parts:
        idx = to_neighs[off:off + b_part].T.reshape(-1)
        embs.append(_sc_gather(features, idx).reshape(k_fan, b_part, f_dim))
        off += b_part
    outs = [_attn_call(e, W11, W12, W21, W22, W31, W32) for e in embs]
    return jnp.concatenate(outs, axis=0)


# speedup vs baseline: 1.0229x; 1.0229x over previous
import functools

import jax
import jax.numpy as jnp
from jax import lax
from jax.experimental import pallas as pl
from jax.experimental.pallas import tpu as pltpu
from jax.experimental.pallas import tpu_sc as plsc


def _sc_gather(features, idx_flat):
    n_rows = idx_flat.shape[0]
    feat_dim = features.shape[1]
    info = plsc.get_sparse_core_info()
    n_workers = info.num_cores * info.num_subcores
    assert n_rows % n_workers == 0
    per_worker = n_rows // n_workers
    chunk = 128
    while per_worker % chunk:
        chunk //= 2
    n_chunks = per_worker // chunk
    num_cores = info.num_cores

    mesh = plsc.VectorSubcoreMesh(core_axis_name="c", subcore_axis_name="s")

    @functools.partial(
        pl.kernel,
        out_type=jax.ShapeDtypeStruct((n_rows, feat_dim), features.dtype),
        mesh=mesh,
        scratch_types=[
            pltpu.VMEM((2, chunk), jnp.int32),
            pltpu.VMEM((2, chunk, feat_dim), jnp.float32),
            pltpu.SemaphoreType.DMA,
            pltpu.SemaphoreType.DMA,
        ],
    )
    def gather_kernel(table_hbm, idx_hbm, out_hbm, idx_v, rows_v, sem0, sem1):
        wid = lax.axis_index("s") * num_cores + lax.axis_index("c")
        base = wid * per_worker
        sems = [sem0, sem1]

        def start(c):
            buf = c % 2
            off = base + c * chunk
            pltpu.sync_copy(idx_hbm.at[pl.ds(off, chunk)], idx_v.at[buf])
            return pltpu.async_copy(
                table_hbm.at[idx_v.at[buf]], rows_v.at[buf], sems[buf])

        pending = start(0)
        for c in range(n_chunks):
            nxt = start(c + 1) if c + 1 < n_chunks else None
            pending.wait()
            pltpu.sync_copy(rows_v.at[c % 2],
                            out_hbm.at[pl.ds(base + c * chunk, chunk)])
            pending = nxt

    return gather_kernel(features, idx_flat)


def _attn_body(emb_ref, w11_ref, w12_ref, w21_ref, w22_ref, w31_ref,
               w32_ref, out_ref):
    x = emb_ref[...]
    k_fan, b_blk, f_dim = x.shape
    xT = jnp.transpose(x.reshape(k_fan * b_blk, f_dim))

    def matT(w_ref, rhs):
        return jax.lax.dot_general(w_ref[...], rhs, (((0,), (0,)), ((), ())))

    u2T = jnp.tanh(matT(w11_ref, xT))
    g = jax.lax.dot_general(
        w22_ref[...], w12_ref[...], (((1,), (1,)), ((), ())))
    t2T = g @ u2T
    t3T = jnp.tanh(matT(w21_ref, xT))
    t4T = matT(w32_ref, jnp.tanh(matT(w31_ref, xT)))

    t2s = [t2T[:, k * b_blk:(k + 1) * b_blk] for k in range(k_fan)]
    t3s = [t3T[:, k * b_blk:(k + 1) * b_blk] for k in range(k_fan)]
    t4s = [t4T[:, k * b_blk:(k + 1) * b_blk] for k in range(k_fan)]

    w = [jnp.zeros((b_blk,), jnp.float32) for _ in range(k_fan)]
    for k in range(k_fan):
        s = [jnp.sum(t2s[k] * t3s[j], axis=0) for j in range(k_fan)]
        m = s[0]
        for j in range(1, k_fan):
            m = jnp.maximum(m, s[j])
        e = [jnp.exp(s[j] - m) for j in range(k_fan)]
        tot = e[0]
        for j in range(1, k_fan):
            tot = tot + e[j]
        inv = 1.0 / tot
        for j in range(k_fan):
            w[j] = w[j] + e[j] * inv

    outT = w[0][None, :] * t4s[0]
    for j in range(1, k_fan):
        outT = outT + w[j][None, :] * t4s[j]
    out_ref[...] = jnp.transpose(outT)


def _attn_call(emb_t, W11, W12, W21, W22, W31, W32, b_blk=512):
    k_fan, batch, f_dim = emb_t.shape
    h_dim = W12.shape[1]
    grid = (batch // b_blk,)
    wspec = pl.BlockSpec((f_dim, h_dim), lambda i: (0, 0))
    return pl.pallas_call(
        _attn_body,
        grid=grid,
        in_specs=[
            pl.BlockSpec((k_fan, b_blk, f_dim), lambda i: (0, i, 0)),
            wspec, wspec, wspec, wspec, wspec, wspec,
        ],
        out_specs=pl.BlockSpec((b_blk, h_dim), lambda i: (i, 0)),
        out_shape=jax.ShapeDtypeStruct((batch, h_dim), jnp.float32),
    )(emb_t, W11, W12, W21, W22, W31, W32)


def kernel(nodes, to_neighs, features, W11, W12, W21, W22, W31, W32):
    del nodes
    batch, k_fan = to_neighs.shape
    f_dim = features.shape[1]
    parts = [batch // 4, batch // 4, batch // 2]
    embs, off = [], 0
    for b_part in parts:
        idx = to_neighs[off:off + b_part].T.reshape(-1)
        embs.append(_sc_gather(features, idx).reshape(k_fan, b_part, f_dim))
        off += b_part
    outs = [_attn_call(e, W11, W12, W21, W22, W31, W32) for e in embs]
    return jnp.concatenate(outs, axis=0)


# speedup vs baseline: 1.0605x; 1.0368x over previous
import functools

import jax
import jax.numpy as jnp
from jax import lax
from jax.experimental import pallas as pl
from jax.experimental.pallas import tpu as pltpu
from jax.experimental.pallas import tpu_sc as plsc


def _sc_gather(features, idx_flat):
    n_rows = idx_flat.shape[0]
    feat_dim = features.shape[1]
    info = plsc.get_sparse_core_info()
    n_workers = info.num_cores * info.num_subcores
    assert n_rows % n_workers == 0
    per_worker = n_rows // n_workers
    chunk = 128
    while per_worker % chunk:
        chunk //= 2
    n_chunks = per_worker // chunk
    num_cores = info.num_cores

    mesh = plsc.VectorSubcoreMesh(core_axis_name="c", subcore_axis_name="s")

    @functools.partial(
        pl.kernel,
        out_type=jax.ShapeDtypeStruct((n_rows, feat_dim), features.dtype),
        mesh=mesh,
        scratch_types=[
            pltpu.VMEM((2, chunk), jnp.int32),
            pltpu.VMEM((2, chunk, feat_dim), jnp.float32),
            pltpu.SemaphoreType.DMA,
            pltpu.SemaphoreType.DMA,
        ],
    )
    def gather_kernel(table_hbm, idx_hbm, out_hbm, idx_v, rows_v, sem0, sem1):
        wid = lax.axis_index("s") * num_cores + lax.axis_index("c")
        base = wid * per_worker
        sems = [sem0, sem1]

        def start(c):
            buf = c % 2
            off = base + c * chunk
            pltpu.sync_copy(idx_hbm.at[pl.ds(off, chunk)], idx_v.at[buf])
            return pltpu.async_copy(
                table_hbm.at[idx_v.at[buf]], rows_v.at[buf], sems[buf])

        pending = start(0)
        for c in range(n_chunks):
            nxt = start(c + 1) if c + 1 < n_chunks else None
            pending.wait()
            pltpu.sync_copy(rows_v.at[c % 2],
                            out_hbm.at[pl.ds(base + c * chunk, chunk)])
            pending = nxt

    return gather_kernel(features, idx_flat)


def _attn_body(emb_ref, w11_ref, w12_ref, w21_ref, w22_ref, w31_ref,
               w32_ref, out_ref):
    x = emb_ref[...]
    k_fan, b_blk, f_dim = x.shape
    xT = jnp.transpose(x.reshape(k_fan * b_blk, f_dim))

    def matT(w_ref, rhs):
        return jax.lax.dot_general(w_ref[...], rhs, (((0,), (0,)), ((), ())))

    u2T = jnp.tanh(matT(w11_ref, xT))
    g = jax.lax.dot_general(
        w22_ref[...], w12_ref[...], (((1,), (1,)), ((), ())))
    t2T = g @ u2T
    t3T = jnp.tanh(matT(w21_ref, xT))
    t4T = matT(w32_ref, jnp.tanh(matT(w31_ref, xT)))

    t2s = [t2T[:, k * b_blk:(k + 1) * b_blk] for k in range(k_fan)]
    t3s = [t3T[:, k * b_blk:(k + 1) * b_blk] for k in range(k_fan)]
    t4s = [t4T[:, k * b_blk:(k + 1) * b_blk] for k in range(k_fan)]

    w = [jnp.zeros((b_blk,), jnp.float32) for _ in range(k_fan)]
    for k in range(k_fan):
        s = [jnp.sum(t2s[k] * t3s[j], axis=0) for j in range(k_fan)]
        m = s[0]
        for j in range(1, k_fan):
            m = jnp.maximum(m, s[j])
        e = [jnp.exp(s[j] - m) for j in range(k_fan)]
        tot = e[0]
        for j in range(1, k_fan):
            tot = tot + e[j]
        inv = 1.0 / tot
        for j in range(k_fan):
            w[j] = w[j] + e[j] * inv

    outT = w[0][None, :] * t4s[0]
    for j in range(1, k_fan):
        outT = outT + w[j][None, :] * t4s[j]
    out_ref[...] = jnp.transpose(outT)


def _attn_call(emb_t, W11, W12, W21, W22, W31, W32, b_blk=512):
    k_fan, batch, f_dim = emb_t.shape
    h_dim = W12.shape[1]
    grid = (batch // b_blk,)
    wspec = pl.BlockSpec((f_dim, h_dim), lambda i: (0, 0))
    return pl.pallas_call(
        _attn_body,
        grid=grid,
        in_specs=[
            pl.BlockSpec((k_fan, b_blk, f_dim), lambda i: (0, i, 0)),
            wspec, wspec, wspec, wspec, wspec, wspec,
        ],
        out_specs=pl.BlockSpec((b_blk, h_dim), lambda i: (i, 0)),
        out_shape=jax.ShapeDtypeStruct((batch, h_dim), jnp.float32),
    )(emb_t, W11, W12, W21, W22, W31, W32)


def kernel(nodes, to_neighs, features, W11, W12, W21, W22, W31, W32):
    del nodes
    batch, k_fan = to_neighs.shape
    f_dim = features.shape[1]
    n_split = 2
    b_part = batch // n_split
    embs = []
    for p in range(n_split):
        idx = to_neighs[p * b_part:(p + 1) * b_part].T.reshape(-1)
        embs.append(_sc_gather(features, idx).reshape(k_fan, b_part, f_dim))
    outs = [_attn_call(e, W11, W12, W21, W22, W31, W32) for e in embs]
    return jnp.concatenate(outs, axis=0)
